# baseline (speedup 1.0000x reference)
import functools

import jax
import jax.numpy as jnp
from jax import lax
from jax.experimental import pallas as pl
from jax.experimental.pallas import tpu as pltpu
from jax.experimental.pallas import tpu_sc as plsc

_K = 32
_N1 = 10000
_N2 = 20000
_NW = 32
_QPW = 320
_NQPAD = _NW * _QPW
_QH = _QPW // 2
_R2 = 0.1 * 0.1
_N2P = 20480
_PPW = _N2P // 16
_CAND = 2048
_INF = 0x7FFFFFFF


def _ball_query_sc(p1x, p1y, p1z, p2x, p2y, p2z):
    f32 = jnp.float32
    i32 = jnp.int32
    mesh = plsc.VectorSubcoreMesh(core_axis_name="c", subcore_axis_name="s")

    @functools.partial(
        pl.kernel,
        out_type=[
            jax.ShapeDtypeStruct((_NQPAD, _K), i32),
            jax.ShapeDtypeStruct((_NQPAD,), i32),
            jax.ShapeDtypeStruct((_NQPAD, _K), f32),
            jax.ShapeDtypeStruct((_NQPAD, _K), f32),
            jax.ShapeDtypeStruct((_NQPAD, _K), f32),
        ],
        mesh=mesh,
        compiler_params=pltpu.CompilerParams(needs_layout_passes=False,
                                             use_tc_tiling_on_sc=False),
        scratch_types=[
            pltpu.VMEM((_N2P + 32,), f32),
            pltpu.VMEM((_N2P + 32,), f32),
            pltpu.VMEM((_N2P + 32,), f32),
            pltpu.VMEM((_N2P + 32,), i32),
            pltpu.VMEM((1024,), i32),
            pltpu.VMEM((1024,), i32),
            pltpu.VMEM((_PPW,), f32),
            pltpu.VMEM((_PPW,), f32),
            pltpu.VMEM((_PPW,), f32),
            pltpu.VMEM((_PPW,), f32),
            pltpu.VMEM((_PPW,), f32),
            pltpu.VMEM((_PPW,), f32),
            pltpu.SemaphoreType.DMA,
            pltpu.SemaphoreType.DMA,
            pltpu.VMEM((_QPW,), f32),
            pltpu.VMEM((_QPW,), f32),
            pltpu.VMEM((_QPW,), f32),
            pltpu.VMEM((_CAND,), i32),
            pltpu.VMEM((_QH, _K), i32),
            pltpu.VMEM((_QPW,), i32),
            pltpu.VMEM((_QH, _K), f32),
            pltpu.VMEM((_QH, _K), f32),
            pltpu.VMEM((_QH, _K), f32),
        ],
    )
    def body(p1x_h, p1y_h, p1z_h, p2x_h, p2y_h, p2z_h,
             map_h, nn_h, ox_h, oy_h, oz_h,
             sx_v, sy_v, sz_v, comb_v, cstart_v, hist_v,
             px_s0, py_s0, pz_s0, px_s1, py_s1, pz_s1, sem0, sem1,
             qx_v, qy_v, qz_v,
             cand_v, map_v, nn_v, ox_v, oy_v, oz_v):
        wid = lax.axis_index("c") * 16 + lax.axis_index("s")
        qbase = wid * _QPW
        lanes = lax.iota(i32, 16)
        zeros16 = jnp.zeros((16,), i32)
        onef = jnp.ones((16,), f32)
        zerof = jnp.zeros((16,), f32)
        inf16 = jnp.full((16,), _INF, i32)

        def scalar0(v16):
            return lax.squeeze(lax.slice_in_dim(v16, 0, 1), (0,))

        def scalar_at(v16, r):
            return lax.squeeze(lax.slice_in_dim(v16, r, r + 1), (0,))

        def cell_of(xv, yv, zv):
            cx = jnp.minimum(xv * 10.0, 9.0).astype(i32)
            cy = jnp.minimum(yv * 10.0, 9.0).astype(i32)
            cz = jnp.minimum(zv * 10.0, 9.0).astype(i32)
            return cx * 100 + cy * 10 + cz, cx, cy, cz

        def zero_hist(h, carry):
            hist_v[pl.ds(h * 16, 16)] = zeros16
            return carry

        lax.fori_loop(0, 64, zero_hist, 0)

        bufs = ((px_s0, py_s0, pz_s0), (px_s1, py_s1, pz_s1))
        sems = (sem0, sem1)
        n_slices = _N2P // _PPW

        def issue(s, b):
            return [pltpu.async_copy(src.at[pl.ds(s * _PPW, _PPW)], dst,
                                     sems[b])
                    for src, dst in zip((p2x_h, p2y_h, p2z_h), bufs[b])]

        def run_pass(make_chunk):
            hs = issue(0, 0)
            for s in range(n_slices):
                b = s % 2
                for h in hs:
                    h.wait()
                if s + 1 < n_slices:
                    hs = issue(s + 1, 1 - b)
                bx, by, bz = bufs[b]
                lax.fori_loop(0, _PPW // 16, make_chunk(bx, by, bz, s), 0)

        def p1_chunk(bx, by, bz, s):
            def chunk(c, carry2):
                xv = bx[pl.ds(c * 16, 16)]
                yv = by[pl.ds(c * 16, 16)]
                zv = bz[pl.ds(c * 16, 16)]
                cid, _, _, _ = cell_of(xv, yv, zv)
                rk, is_last = plsc.scan_count(cid)
                old = plsc.load_gather(hist_v, [cid])
                plsc.store_scatter(hist_v, [cid], old + rk, mask=is_last)
                return carry2
            return chunk

        run_pass(p1_chunk)

        def prefix(h, carry):
            ch = hist_v[pl.ds(h * 16, 16)]
            inc = plsc.cumsum(ch)
            cstart_v[pl.ds(h * 16, 16)] = (carry + inc) - ch
            return carry + scalar_at(inc, 15)

        lax.fori_loop(0, 64, prefix, jnp.int32(0))
        lax.fori_loop(0, 64, zero_hist, 0)

        def p2_chunk(bx, by, bz, s):
            def chunk(c, carry2):
                xv = bx[pl.ds(c * 16, 16)]
                yv = by[pl.ds(c * 16, 16)]
                zv = bz[pl.ds(c * 16, 16)]
                cid, _, _, _ = cell_of(xv, yv, zv)
                rk, is_last = plsc.scan_count(cid)
                old = plsc.load_gather(hist_v, [cid])
                base = plsc.load_gather(cstart_v, [cid])
                pos = (base + old) + (rk - 1)
                gidx = (s * _PPW + c * 16) + lanes
                plsc.store_scatter(hist_v, [cid], old + rk, mask=is_last)
                plsc.store_scatter(sx_v, [pos], xv)
                plsc.store_scatter(sy_v, [pos], yv)
                plsc.store_scatter(sz_v, [pos], zv)
                plsc.store_scatter(comb_v, [pos], (gidx << 15) | pos)
                return carry2
            return chunk

        run_pass(p2_chunk)

        pltpu.sync_copy(p1x_h.at[pl.ds(qbase, _QPW)], qx_v)
        pltpu.sync_copy(p1y_h.at[pl.ds(qbase, _QPW)], qy_v)
        pltpu.sync_copy(p1z_h.at[pl.ds(qbase, _QPW)], qz_v)

        dxv = lanes // 3 - 1
        dyv = lanes % 3 - 1
        lane_lt9 = lanes < 9

        for half in range(2):
            def per_query(i, carry):
                qi = half * _QH + i
                isplat = jnp.full((16,), qi, i32)
                qx = plsc.load_gather(qx_v, [isplat])
                qy = plsc.load_gather(qy_v, [isplat])
                qz = plsc.load_gather(qz_v, [isplat])
                _, cx, cy, cz = cell_of(qx, qy, qz)
                cz0 = jnp.maximum(cz - 1, 0)
                cz1 = jnp.minimum(cz + 1, 9)
                rowx = cx + dxv
                rowy = cy + dyv
                okrow = ((rowx >= 0) & (rowx <= 9) & (rowy >= 0)
                         & (rowy <= 9) & lane_lt9)
                okrow = okrow & jnp.full((16,), qbase + qi < _N1, jnp.bool_)
                cidr = rowx * 100 + rowy * 10
                cid0 = jnp.where(okrow, cidr + cz0, 0)
                cid1p = jnp.where(okrow, (cidr + cz1) + 1, 0)
                sv = plsc.load_gather(cstart_v, [cid0])
                ev = plsc.load_gather(cstart_v, [cid1p])
                lenv = jnp.where(okrow, ev - sv, 0)

                def seg_chunk(st_r, ln_r):
                    def chunkq(c, cnt):
                        for u in range(2):
                            off = st_r + c * 32 + u * 16
                            sxv = sx_v[pl.ds(off, 16)]
                            syv = sy_v[pl.ds(off, 16)]
                            szv = sz_v[pl.ds(off, 16)]
                            comb = comb_v[pl.ds(off, 16)]
                            lm = (lanes + (c * 32 + u * 16)) < ln_r
                            dx = qx - sxv
                            dy = qy - syv
                            dz = qz - szv
                            d2 = dx * dx + dy * dy
                            d2 = d2 + dz * dz
                            within = (d2 <= _R2) & lm
                            cntc = jnp.minimum(cnt, _CAND - 16)
                            plsc.store_compressed(cand_v.at[pl.ds(cntc, 16)],
                                                  comb, mask=within)
                            c16 = plsc.all_reduce_population_count(within)
                            cnt = cnt + scalar0(c16)
                        return cnt
                    return chunkq

                cnt = jnp.int32(0)
                for r in range(9):
                    st_r = scalar_at(sv, r)
                    ln_r = scalar_at(lenv, r)
                    nch = (ln_r + 31) // 32
                    cnt = lax.fori_loop(0, nch, seg_chunk(st_r, ln_r), cnt)

                def select(c, b):
                    b0, b1 = b
                    ch = cand_v[pl.ds(c * 16, 16)]
                    ch = jnp.where((lanes + c * 16) < cnt, ch, inf16)
                    ch = lax.sort(ch)
                    rb = lax.rev(ch, (0,))
                    b0n = lax.sort(jnp.minimum(b0, rb))
                    x = lax.sort(jnp.maximum(b0, rb))
                    rx = lax.rev(x, (0,))
                    b1n = lax.sort(jnp.minimum(b1, rx))
                    return b0n, b1n

                ch0 = jnp.where(lanes < cnt, cand_v[pl.ds(0, 16)], inf16)
                b0 = lax.sort(ch0)
                ch1 = jnp.where((lanes + 16) < cnt, cand_v[pl.ds(16, 16)],
                                inf16)
                ch1 = lax.sort(ch1)
                rb = lax.rev(ch1, (0,))
                b1 = lax.sort(jnp.maximum(b0, rb))
                b0 = lax.sort(jnp.minimum(b0, rb))
                nsel = (jnp.minimum(cnt, _CAND) + 15) // 16
                b0, b1 = lax.fori_loop(2, nsel, select, (b0, b1))

                nn_s = jnp.minimum(cnt, _K)
                nn_splat = jnp.full((16,), nn_s, i32)
                plsc.store_scatter(nn_v, [isplat], nn_splat, mask=lanes == 0)
                for cc, b in enumerate((b0, b1)):
                    validm = (lanes + cc * 16) < nn_splat
                    sidxo = jnp.where(validm, b >> 15, 0)
                    poso = jnp.where(validm, b & 32767, 0)
                    vf = jnp.where(validm, onef, zerof)
                    map_v[i, pl.ds(cc * 16, 16)] = sidxo
                    ox_v[i, pl.ds(cc * 16, 16)] = \
                        plsc.load_gather(sx_v, [poso]) * vf
                    oy_v[i, pl.ds(cc * 16, 16)] = \
                        plsc.load_gather(sy_v, [poso]) * vf
                    oz_v[i, pl.ds(cc * 16, 16)] = \
                        plsc.load_gather(sz_v, [poso]) * vf
                return carry

            lax.fori_loop(0, _QH, per_query, 0)
            hb = qbase + half * _QH
            pltpu.sync_copy(map_v, map_h.at[pl.ds(hb, _QH)])
            pltpu.sync_copy(ox_v, ox_h.at[pl.ds(hb, _QH)])
            pltpu.sync_copy(oy_v, oy_h.at[pl.ds(hb, _QH)])
            pltpu.sync_copy(oz_v, oz_h.at[pl.ds(hb, _QH)])
        pltpu.sync_copy(nn_v, nn_h.at[pl.ds(qbase, _QPW)])

    return body(p1x, p1y, p1z, p2x, p2y, p2z)


def kernel(points1, points2):
    p1 = points1[0]
    p2 = points2[0]
    p1p = jnp.pad(p1, ((0, _NQPAD - _N1), (0, 0)))
    p2p = jnp.pad(p2, ((0, _N2P - _N2), (0, 0)), constant_values=1e6)
    p1x, p1y, p1z = p1p[:, 0], p1p[:, 1], p1p[:, 2]
    p2x, p2y, p2z = p2p[:, 0], p2p[:, 1], p2p[:, 2]
    mp, nn, ox, oy, oz = _ball_query_sc(p1x, p1y, p1z, p2x, p2y, p2z)
    mapping = mp[:_N1].reshape(1, _N1, _K)
    num_neighbors = nn[:_N1].reshape(1, _N1)
    outputs = jnp.stack([ox[:_N1], oy[:_N1], oz[:_N1]], axis=-1)
    outputs = outputs.reshape(1, _N1, _K, 3)
    return mapping, num_neighbors, outputs


# speedup vs baseline: 1.0516x; 1.0516x over previous
import functools

import jax
import jax.numpy as jnp
from jax import lax
from jax.experimental import pallas as pl
from jax.experimental.pallas import tpu as pltpu
from jax.experimental.pallas import tpu_sc as plsc

_K = 32
_N1 = 10000
_N2 = 20000
_NW = 32
_QPW = 320
_NQPAD = _NW * _QPW
_QH = _QPW // 2
_R2 = 0.1 * 0.1
_N2P = 20480
_PPW = _N2P // 16
_CAND = 2048
_INF = 0x7FFFFFFF


def _ball_query_sc(p1x, p1y, p1z, p2x, p2y, p2z):
    f32 = jnp.float32
    i32 = jnp.int32
    mesh = plsc.VectorSubcoreMesh(core_axis_name="c", subcore_axis_name="s")

    @functools.partial(
        pl.kernel,
        out_type=[
            jax.ShapeDtypeStruct((_NQPAD, _K), i32),
            jax.ShapeDtypeStruct((_NQPAD,), i32),
            jax.ShapeDtypeStruct((_NQPAD, _K), f32),
            jax.ShapeDtypeStruct((_NQPAD, _K), f32),
            jax.ShapeDtypeStruct((_NQPAD, _K), f32),
        ],
        mesh=mesh,
        compiler_params=pltpu.CompilerParams(needs_layout_passes=False,
                                             use_tc_tiling_on_sc=False),
        scratch_types=[
            pltpu.VMEM((_N2P + 32,), f32),
            pltpu.VMEM((_N2P + 32,), f32),
            pltpu.VMEM((_N2P + 32,), f32),
            pltpu.VMEM((_N2P + 32,), i32),
            pltpu.VMEM((1024,), i32),
            pltpu.VMEM((1024,), i32),
            pltpu.VMEM((_PPW,), f32),
            pltpu.VMEM((_PPW,), f32),
            pltpu.VMEM((_PPW,), f32),
            pltpu.VMEM((_PPW,), f32),
            pltpu.VMEM((_PPW,), f32),
            pltpu.VMEM((_PPW,), f32),
            pltpu.SemaphoreType.DMA,
            pltpu.SemaphoreType.DMA,
            pltpu.VMEM((_QPW,), f32),
            pltpu.VMEM((_QPW,), f32),
            pltpu.VMEM((_QPW,), f32),
            pltpu.VMEM((_CAND,), i32),
            pltpu.VMEM((_QH, _K), i32),
            pltpu.VMEM((_QPW,), i32),
            pltpu.VMEM((_QH, _K), f32),
            pltpu.VMEM((_QH, _K), f32),
            pltpu.VMEM((_QH, _K), f32),
        ],
    )
    def body(p1x_h, p1y_h, p1z_h, p2x_h, p2y_h, p2z_h,
             map_h, nn_h, ox_h, oy_h, oz_h,
             sx_v, sy_v, sz_v, comb_v, cstart_v, hist_v,
             px_s0, py_s0, pz_s0, px_s1, py_s1, pz_s1, sem0, sem1,
             qx_v, qy_v, qz_v,
             cand_v, map_v, nn_v, ox_v, oy_v, oz_v):
        wid = lax.axis_index("c") * 16 + lax.axis_index("s")
        qbase = wid * _QPW
        lanes = lax.iota(i32, 16)
        zeros16 = jnp.zeros((16,), i32)
        onef = jnp.ones((16,), f32)
        zerof = jnp.zeros((16,), f32)
        inf16 = jnp.full((16,), _INF, i32)

        def scalar0(v16):
            return lax.squeeze(lax.slice_in_dim(v16, 0, 1), (0,))

        def scalar_at(v16, r):
            return lax.squeeze(lax.slice_in_dim(v16, r, r + 1), (0,))

        def cell_of(xv, yv, zv):
            cx = jnp.minimum(xv * 10.0, 9.0).astype(i32)
            cy = jnp.minimum(yv * 10.0, 9.0).astype(i32)
            cz = jnp.minimum(zv * 10.0, 9.0).astype(i32)
            return cx * 100 + cy * 10 + cz, cx, cy, cz

        def zero_hist(h, carry):
            hist_v[pl.ds(h * 16, 16)] = zeros16
            return carry

        lax.fori_loop(0, 64, zero_hist, 0)

        bufs = ((px_s0, py_s0, pz_s0), (px_s1, py_s1, pz_s1))
        sems = (sem0, sem1)
        n_slices = _N2P // _PPW

        def issue(s, b):
            return [pltpu.async_copy(src.at[pl.ds(s * _PPW, _PPW)], dst,
                                     sems[b])
                    for src, dst in zip((p2x_h, p2y_h, p2z_h), bufs[b])]

        def run_pass(make_chunk):
            hs = issue(0, 0)
            for s in range(n_slices):
                b = s % 2
                for h in hs:
                    h.wait()
                if s + 1 < n_slices:
                    hs = issue(s + 1, 1 - b)
                bx, by, bz = bufs[b]
                lax.fori_loop(0, _PPW // 16, make_chunk(bx, by, bz, s), 0)

        def p1_chunk(bx, by, bz, s):
            def chunk(c, carry2):
                xv = bx[pl.ds(c * 16, 16)]
                yv = by[pl.ds(c * 16, 16)]
                zv = bz[pl.ds(c * 16, 16)]
                cid, _, _, _ = cell_of(xv, yv, zv)
                rk, is_last = plsc.scan_count(cid)
                old = plsc.load_gather(hist_v, [cid])
                plsc.store_scatter(hist_v, [cid], old + rk, mask=is_last)
                return carry2
            return chunk

        run_pass(p1_chunk)

        def prefix(h, carry):
            ch = hist_v[pl.ds(h * 16, 16)]
            inc = plsc.cumsum(ch)
            cstart_v[pl.ds(h * 16, 16)] = (carry + inc) - ch
            return carry + scalar_at(inc, 15)

        lax.fori_loop(0, 64, prefix, jnp.int32(0))
        lax.fori_loop(0, 64, zero_hist, 0)

        def p2_chunk(bx, by, bz, s):
            def chunk(c, carry2):
                xv = bx[pl.ds(c * 16, 16)]
                yv = by[pl.ds(c * 16, 16)]
                zv = bz[pl.ds(c * 16, 16)]
                cid, _, _, _ = cell_of(xv, yv, zv)
                rk, is_last = plsc.scan_count(cid)
                old = plsc.load_gather(hist_v, [cid])
                base = plsc.load_gather(cstart_v, [cid])
                pos = (base + old) + (rk - 1)
                gidx = (s * _PPW + c * 16) + lanes
                plsc.store_scatter(hist_v, [cid], old + rk, mask=is_last)
                plsc.store_scatter(sx_v, [pos], xv)
                plsc.store_scatter(sy_v, [pos], yv)
                plsc.store_scatter(sz_v, [pos], zv)
                plsc.store_scatter(comb_v, [pos], (gidx << 15) | pos)
                return carry2
            return chunk

        run_pass(p2_chunk)

        pltpu.sync_copy(p1x_h.at[pl.ds(qbase, _QPW)], qx_v)
        pltpu.sync_copy(p1y_h.at[pl.ds(qbase, _QPW)], qy_v)
        pltpu.sync_copy(p1z_h.at[pl.ds(qbase, _QPW)], qz_v)

        dxv = lanes // 3 - 1
        dyv = lanes % 3 - 1
        lane_lt9 = lanes < 9

        for half in range(2):
            def per_query(i, carry):
                qi = half * _QH + i
                isplat = jnp.full((16,), qi, i32)
                qx = plsc.load_gather(qx_v, [isplat])
                qy = plsc.load_gather(qy_v, [isplat])
                qz = plsc.load_gather(qz_v, [isplat])
                _, cx, cy, cz = cell_of(qx, qy, qz)
                cz0 = jnp.maximum(cz - 1, 0)
                cz1 = jnp.minimum(cz + 1, 9)
                rowx = cx + dxv
                rowy = cy + dyv
                okrow = ((rowx >= 0) & (rowx <= 9) & (rowy >= 0)
                         & (rowy <= 9) & lane_lt9)
                okrow = okrow & jnp.full((16,), qbase + qi < _N1, jnp.bool_)
                cidr = rowx * 100 + rowy * 10
                cid0 = jnp.where(okrow, cidr + cz0, 0)
                cid1p = jnp.where(okrow, (cidr + cz1) + 1, 0)
                sv = plsc.load_gather(cstart_v, [cid0])
                ev = plsc.load_gather(cstart_v, [cid1p])
                lenv = jnp.where(okrow, ev - sv, 0)

                def seg_chunk(st_r, ln_r):
                    def chunkq(c, cnt):
                        off = st_r + c * 16
                        sxv = sx_v[pl.ds(off, 16)]
                        syv = sy_v[pl.ds(off, 16)]
                        szv = sz_v[pl.ds(off, 16)]
                        comb = comb_v[pl.ds(off, 16)]
                        lm = (lanes + c * 16) < ln_r
                        dx = qx - sxv
                        dy = qy - syv
                        dz = qz - szv
                        d2 = dx * dx + dy * dy
                        d2 = d2 + dz * dz
                        within = (d2 <= _R2) & lm
                        cntc = jnp.minimum(cnt, _CAND - 16)
                        plsc.store_compressed(cand_v.at[pl.ds(cntc, 16)],
                                              comb, mask=within)
                        c16 = plsc.all_reduce_population_count(within)
                        return cnt + scalar0(c16)
                    return chunkq

                cnt = jnp.int32(0)
                for r in range(9):
                    st_r = scalar_at(sv, r)
                    ln_r = scalar_at(lenv, r)
                    nch = (ln_r + 15) // 16
                    cnt = lax.fori_loop(0, nch, seg_chunk(st_r, ln_r), cnt)

                def select(c, b):
                    b0, b1 = b
                    ch = cand_v[pl.ds(c * 16, 16)]
                    ch = jnp.where((lanes + c * 16) < cnt, ch, inf16)
                    ch = lax.sort(ch)
                    rb = lax.rev(ch, (0,))
                    b0n = lax.sort(jnp.minimum(b0, rb))
                    x = lax.sort(jnp.maximum(b0, rb))
                    rx = lax.rev(x, (0,))
                    b1n = lax.sort(jnp.minimum(b1, rx))
                    return b0n, b1n

                ch0 = jnp.where(lanes < cnt, cand_v[pl.ds(0, 16)], inf16)
                b0 = lax.sort(ch0)
                ch1 = jnp.where((lanes + 16) < cnt, cand_v[pl.ds(16, 16)],
                                inf16)
                ch1 = lax.sort(ch1)
                rb = lax.rev(ch1, (0,))
                b1 = lax.sort(jnp.maximum(b0, rb))
                b0 = lax.sort(jnp.minimum(b0, rb))
                nsel = (jnp.minimum(cnt, _CAND) + 15) // 16
                b0, b1 = lax.fori_loop(2, nsel, select, (b0, b1))

                nn_s = jnp.minimum(cnt, _K)
                nn_splat = jnp.full((16,), nn_s, i32)
                plsc.store_scatter(nn_v, [isplat], nn_splat, mask=lanes == 0)
                for cc, b in enumerate((b0, b1)):
                    validm = (lanes + cc * 16) < nn_splat
                    sidxo = jnp.where(validm, b >> 15, 0)
                    poso = jnp.where(validm, b & 32767, 0)
                    vf = jnp.where(validm, onef, zerof)
                    map_v[i, pl.ds(cc * 16, 16)] = sidxo
                    ox_v[i, pl.ds(cc * 16, 16)] = \
                        plsc.load_gather(sx_v, [poso]) * vf
                    oy_v[i, pl.ds(cc * 16, 16)] = \
                        plsc.load_gather(sy_v, [poso]) * vf
                    oz_v[i, pl.ds(cc * 16, 16)] = \
                        plsc.load_gather(sz_v, [poso]) * vf
                return carry

            lax.fori_loop(0, _QH, per_query, 0)
            hb = qbase + half * _QH
            pltpu.sync_copy(map_v, map_h.at[pl.ds(hb, _QH)])
            pltpu.sync_copy(ox_v, ox_h.at[pl.ds(hb, _QH)])
            pltpu.sync_copy(oy_v, oy_h.at[pl.ds(hb, _QH)])
            pltpu.sync_copy(oz_v, oz_h.at[pl.ds(hb, _QH)])
        pltpu.sync_copy(nn_v, nn_h.at[pl.ds(qbase, _QPW)])

    return body(p1x, p1y, p1z, p2x, p2y, p2z)


def kernel(points1, points2):
    p1 = points1[0]
    p2 = points2[0]
    p1p = jnp.pad(p1, ((0, _NQPAD - _N1), (0, 0)))
    p2p = jnp.pad(p2, ((0, _N2P - _N2), (0, 0)), constant_values=1e6)
    p1x, p1y, p1z = p1p[:, 0], p1p[:, 1], p1p[:, 2]
    p2x, p2y, p2z = p2p[:, 0], p2p[:, 1], p2p[:, 2]
    mp, nn, ox, oy, oz = _ball_query_sc(p1x, p1y, p1z, p2x, p2y, p2z)
    mapping = mp[:_N1].reshape(1, _N1, _K)
    num_neighbors = nn[:_N1].reshape(1, _N1)
    outputs = jnp.stack([ox[:_N1], oy[:_N1], oz[:_N1]], axis=-1)
    outputs = outputs.reshape(1, _N1, _K, 3)
    return mapping, num_neighbors, outputs


# speedup vs baseline: 1.1190x; 1.0641x over previous
import functools

import jax
import jax.numpy as jnp
from jax import lax
from jax.experimental import pallas as pl
from jax.experimental.pallas import tpu as pltpu
from jax.experimental.pallas import tpu_sc as plsc

_K = 32
_N1 = 10000
_N2 = 20000
_NW = 32
_QPW = 320
_NQPAD = _NW * _QPW
_QH = _QPW // 2
_R2 = 0.1 * 0.1
_N2P = 20480
_PPW = _N2P // 16
_CAND = 2048
_INF = 0x7FFFFFFF


def _ball_query_sc(p1x, p1y, p1z, p2x, p2y, p2z):
    f32 = jnp.float32
    i32 = jnp.int32
    mesh = plsc.VectorSubcoreMesh(core_axis_name="c", subcore_axis_name="s")

    @functools.partial(
        pl.kernel,
        out_type=[
            jax.ShapeDtypeStruct((_NQPAD, _K), i32),
            jax.ShapeDtypeStruct((_NQPAD,), i32),
            jax.ShapeDtypeStruct((_NQPAD, _K), f32),
            jax.ShapeDtypeStruct((_NQPAD, _K), f32),
            jax.ShapeDtypeStruct((_NQPAD, _K), f32),
        ],
        mesh=mesh,
        compiler_params=pltpu.CompilerParams(needs_layout_passes=False,
                                             use_tc_tiling_on_sc=False),
        scratch_types=[
            pltpu.VMEM((_N2P + 32,), f32),
            pltpu.VMEM((_N2P + 32,), f32),
            pltpu.VMEM((_N2P + 32,), f32),
            pltpu.VMEM((_N2P + 32,), i32),
            pltpu.VMEM((1024,), i32),
            pltpu.VMEM((1024,), i32),
            pltpu.VMEM((_PPW,), f32),
            pltpu.VMEM((_PPW,), f32),
            pltpu.VMEM((_PPW,), f32),
            pltpu.VMEM((_PPW,), f32),
            pltpu.VMEM((_PPW,), f32),
            pltpu.VMEM((_PPW,), f32),
            pltpu.SemaphoreType.DMA,
            pltpu.SemaphoreType.DMA,
            pltpu.VMEM((_QPW,), f32),
            pltpu.VMEM((_QPW,), f32),
            pltpu.VMEM((_QPW,), f32),
            pltpu.VMEM((_CAND,), i32),
            pltpu.VMEM((_QH, _K), i32),
            pltpu.VMEM((_QPW,), i32),
            pltpu.VMEM((_QH, _K), f32),
            pltpu.VMEM((_QH, _K), f32),
            pltpu.VMEM((_QH, _K), f32),
        ],
    )
    def body(p1x_h, p1y_h, p1z_h, p2x_h, p2y_h, p2z_h,
             map_h, nn_h, ox_h, oy_h, oz_h,
             sx_v, sy_v, sz_v, comb_v, cstart_v, hist_v,
             px_s0, py_s0, pz_s0, px_s1, py_s1, pz_s1, sem0, sem1,
             qx_v, qy_v, qz_v,
             cand_v, map_v, nn_v, ox_v, oy_v, oz_v):
        wid = lax.axis_index("c") * 16 + lax.axis_index("s")
        qbase = wid * _QPW
        lanes = lax.iota(i32, 16)
        zeros16 = jnp.zeros((16,), i32)
        onef = jnp.ones((16,), f32)
        zerof = jnp.zeros((16,), f32)
        inf16 = jnp.full((16,), _INF, i32)

        def scalar0(v16):
            return lax.squeeze(lax.slice_in_dim(v16, 0, 1), (0,))

        def scalar_at(v16, r):
            return lax.squeeze(lax.slice_in_dim(v16, r, r + 1), (0,))

        def cell_of(xv, yv, zv):
            cx = jnp.minimum(xv * 10.0, 9.0).astype(i32)
            cy = jnp.minimum(yv * 10.0, 9.0).astype(i32)
            cz = jnp.minimum(zv * 10.0, 9.0).astype(i32)
            return cx * 100 + cy * 10 + cz, cx, cy, cz

        def zero_hist(h, carry):
            hist_v[pl.ds(h * 16, 16)] = zeros16
            return carry

        lax.fori_loop(0, 64, zero_hist, 0)

        bufs = ((px_s0, py_s0, pz_s0), (px_s1, py_s1, pz_s1))
        sems = (sem0, sem1)
        n_slices = _N2P // _PPW

        def issue(s, b):
            return [pltpu.async_copy(src.at[pl.ds(s * _PPW, _PPW)], dst,
                                     sems[b])
                    for src, dst in zip((p2x_h, p2y_h, p2z_h), bufs[b])]

        def run_pass(make_chunk):
            hs = issue(0, 0)
            for s in range(n_slices):
                b = s % 2
                for h in hs:
                    h.wait()
                if s + 1 < n_slices:
                    hs = issue(s + 1, 1 - b)
                bx, by, bz = bufs[b]
                lax.fori_loop(0, _PPW // 16, make_chunk(bx, by, bz, s), 0)

        def p1_chunk(bx, by, bz, s):
            def chunk(c, carry2):
                xv = bx[pl.ds(c * 16, 16)]
                yv = by[pl.ds(c * 16, 16)]
                zv = bz[pl.ds(c * 16, 16)]
                cid, _, _, _ = cell_of(xv, yv, zv)
                rk, is_last = plsc.scan_count(cid)
                old = plsc.load_gather(hist_v, [cid])
                plsc.store_scatter(hist_v, [cid], old + rk, mask=is_last)
                return carry2
            return chunk

        run_pass(p1_chunk)

        def prefix(h, carry):
            ch = hist_v[pl.ds(h * 16, 16)]
            inc = plsc.cumsum(ch)
            cstart_v[pl.ds(h * 16, 16)] = (carry + inc) - ch
            return carry + scalar_at(inc, 15)

        lax.fori_loop(0, 64, prefix, jnp.int32(0))
        lax.fori_loop(0, 64, zero_hist, 0)

        def p2_chunk(bx, by, bz, s):
            def chunk(c, carry2):
                xv = bx[pl.ds(c * 16, 16)]
                yv = by[pl.ds(c * 16, 16)]
                zv = bz[pl.ds(c * 16, 16)]
                cid, _, _, _ = cell_of(xv, yv, zv)
                rk, is_last = plsc.scan_count(cid)
                old = plsc.load_gather(hist_v, [cid])
                base = plsc.load_gather(cstart_v, [cid])
                pos = (base + old) + (rk - 1)
                gidx = (s * _PPW + c * 16) + lanes
                plsc.store_scatter(hist_v, [cid], old + rk, mask=is_last)
                plsc.store_scatter(sx_v, [pos], xv)
                plsc.store_scatter(sy_v, [pos], yv)
                plsc.store_scatter(sz_v, [pos], zv)
                plsc.store_scatter(comb_v, [pos], (gidx << 15) | pos)
                return carry2
            return chunk

        run_pass(p2_chunk)

        pltpu.sync_copy(p1x_h.at[pl.ds(qbase, _QPW)], qx_v)
        pltpu.sync_copy(p1y_h.at[pl.ds(qbase, _QPW)], qy_v)
        pltpu.sync_copy(p1z_h.at[pl.ds(qbase, _QPW)], qz_v)

        dxv = lanes // 3 - 1
        dyv = lanes % 3 - 1
        lane_lt9 = lanes < 9

        for half in range(2):
            def per_query(i, carry):
                qi = half * _QH + i
                isplat = jnp.full((16,), qi, i32)
                qx = plsc.load_gather(qx_v, [isplat])
                qy = plsc.load_gather(qy_v, [isplat])
                qz = plsc.load_gather(qz_v, [isplat])
                _, cx, cy, cz = cell_of(qx, qy, qz)
                rowx = cx + dxv
                rowy = cy + dyv
                colxl = rowx.astype(f32) * 0.1
                dxm = jnp.maximum(jnp.maximum(colxl - qx, qx - (colxl + 0.1)),
                                  0.0)
                colyl = rowy.astype(f32) * 0.1
                dym = jnp.maximum(jnp.maximum(colyl - qy, qy - (colyl + 0.1)),
                                  0.0)
                rem = (_R2 + 1e-5 - dxm * dxm) - dym * dym
                czf = cz.astype(f32) * 0.1
                dzlo = qz - czf
                dzhi = (czf + 0.1) - qz
                cz0 = jnp.maximum(cz - jnp.where(dzlo * dzlo <= rem, 1, 0), 0)
                cz1 = jnp.minimum(cz + jnp.where(dzhi * dzhi <= rem, 1, 0), 9)
                okrow = ((rowx >= 0) & (rowx <= 9) & (rowy >= 0)
                         & (rowy <= 9) & lane_lt9 & (rem >= 0.0))
                okrow = okrow & jnp.full((16,), qbase + qi < _N1, jnp.bool_)
                cidr = rowx * 100 + rowy * 10
                cid0 = jnp.where(okrow, cidr + cz0, 0)
                cid1p = jnp.where(okrow, (cidr + cz1) + 1, 0)
                sv = plsc.load_gather(cstart_v, [cid0])
                ev = plsc.load_gather(cstart_v, [cid1p])
                lenv = jnp.where(okrow, ev - sv, 0)

                def seg_chunk(st_r, ln_r):
                    def chunkq(c, cnt):
                        off = st_r + c * 16
                        sxv = sx_v[pl.ds(off, 16)]
                        syv = sy_v[pl.ds(off, 16)]
                        szv = sz_v[pl.ds(off, 16)]
                        comb = comb_v[pl.ds(off, 16)]
                        lm = (lanes + c * 16) < ln_r
                        dx = qx - sxv
                        dy = qy - syv
                        dz = qz - szv
                        d2 = dx * dx + dy * dy
                        d2 = d2 + dz * dz
                        within = (d2 <= _R2) & lm
                        cntc = jnp.minimum(cnt, _CAND - 16)
                        plsc.store_compressed(cand_v.at[pl.ds(cntc, 16)],
                                              comb, mask=within)
                        c16 = plsc.all_reduce_population_count(within)
                        return cnt + scalar0(c16)
                    return chunkq

                cnt = jnp.int32(0)
                for r in range(9):
                    st_r = scalar_at(sv, r)
                    ln_r = scalar_at(lenv, r)
                    nch = (ln_r + 15) // 16
                    cnt = lax.fori_loop(0, nch, seg_chunk(st_r, ln_r), cnt)

                def select(c, b):
                    b0, b1 = b
                    ch = cand_v[pl.ds(c * 16, 16)]
                    ch = jnp.where((lanes + c * 16) < cnt, ch, inf16)
                    ch = lax.sort(ch)
                    rb = lax.rev(ch, (0,))
                    b0n = lax.sort(jnp.minimum(b0, rb))
                    x = lax.sort(jnp.maximum(b0, rb))
                    rx = lax.rev(x, (0,))
                    b1n = lax.sort(jnp.minimum(b1, rx))
                    return b0n, b1n

                ch0 = jnp.where(lanes < cnt, cand_v[pl.ds(0, 16)], inf16)
                b0 = lax.sort(ch0)
                ch1 = jnp.where((lanes + 16) < cnt, cand_v[pl.ds(16, 16)],
                                inf16)
                ch1 = lax.sort(ch1)
                rb = lax.rev(ch1, (0,))
                b1 = lax.sort(jnp.maximum(b0, rb))
                b0 = lax.sort(jnp.minimum(b0, rb))
                nsel = (jnp.minimum(cnt, _CAND) + 15) // 16
                b0, b1 = lax.fori_loop(2, nsel, select, (b0, b1))

                nn_s = jnp.minimum(cnt, _K)
                nn_splat = jnp.full((16,), nn_s, i32)
                plsc.store_scatter(nn_v, [isplat], nn_splat, mask=lanes == 0)
                for cc, b in enumerate((b0, b1)):
                    validm = (lanes + cc * 16) < nn_splat
                    sidxo = jnp.where(validm, b >> 15, 0)
                    poso = jnp.where(validm, b & 32767, 0)
                    vf = jnp.where(validm, onef, zerof)
                    map_v[i, pl.ds(cc * 16, 16)] = sidxo
                    ox_v[i, pl.ds(cc * 16, 16)] = \
                        plsc.load_gather(sx_v, [poso]) * vf
                    oy_v[i, pl.ds(cc * 16, 16)] = \
                        plsc.load_gather(sy_v, [poso]) * vf
                    oz_v[i, pl.ds(cc * 16, 16)] = \
                        plsc.load_gather(sz_v, [poso]) * vf
                return carry

            lax.fori_loop(0, _QH, per_query, 0)
            hb = qbase + half * _QH
            pltpu.sync_copy(map_v, map_h.at[pl.ds(hb, _QH)])
            pltpu.sync_copy(ox_v, ox_h.at[pl.ds(hb, _QH)])
            pltpu.sync_copy(oy_v, oy_h.at[pl.ds(hb, _QH)])
            pltpu.sync_copy(oz_v, oz_h.at[pl.ds(hb, _QH)])
        pltpu.sync_copy(nn_v, nn_h.at[pl.ds(qbase, _QPW)])

    return body(p1x, p1y, p1z, p2x, p2y, p2z)


def kernel(points1, points2):
    p1 = points1[0]
    p2 = points2[0]
    p1p = jnp.pad(p1, ((0, _NQPAD - _N1), (0, 0)))
    p2p = jnp.pad(p2, ((0, _N2P - _N2), (0, 0)), constant_values=1e6)
    p1x, p1y, p1z = p1p[:, 0], p1p[:, 1], p1p[:, 2]
    p2x, p2y, p2z = p2p[:, 0], p2p[:, 1], p2p[:, 2]
    mp, nn, ox, oy, oz = _ball_query_sc(p1x, p1y, p1z, p2x, p2y, p2z)
    mapping = mp[:_N1].reshape(1, _N1, _K)
    num_neighbors = nn[:_N1].reshape(1, _N1)
    outputs = jnp.stack([ox[:_N1], oy[:_N1], oz[:_N1]], axis=-1)
    outputs = outputs.reshape(1, _N1, _K, 3)
    return mapping, num_neighbors, outputs


# speedup vs baseline: 1.1279x; 1.0079x over previous
import functools

import jax
import jax.numpy as jnp
from jax import lax
from jax.experimental import pallas as pl
from jax.experimental.pallas import tpu as pltpu
from jax.experimental.pallas import tpu_sc as plsc

_K = 32
_N1 = 10000
_N2 = 20000
_NW = 32
_QPW = 320
_NQPAD = _NW * _QPW
_QH = _QPW // 2
_R2 = 0.1 * 0.1
_N2P = 20480
_PPW = _N2P // 16
_CAND = 2048
_INF = 0x7FFFFFFF


def _ball_query_sc(p1x, p1y, p1z, p2x, p2y, p2z):
    f32 = jnp.float32
    i32 = jnp.int32
    mesh = plsc.VectorSubcoreMesh(core_axis_name="c", subcore_axis_name="s")

    @functools.partial(
        pl.kernel,
        out_type=[
            jax.ShapeDtypeStruct((_NQPAD, _K), i32),
            jax.ShapeDtypeStruct((_NQPAD,), i32),
            jax.ShapeDtypeStruct((_NQPAD, _K), f32),
            jax.ShapeDtypeStruct((_NQPAD, _K), f32),
            jax.ShapeDtypeStruct((_NQPAD, _K), f32),
        ],
        mesh=mesh,
        compiler_params=pltpu.CompilerParams(needs_layout_passes=False,
                                             use_tc_tiling_on_sc=False),
        scratch_types=[
            pltpu.VMEM((_N2P + 32,), f32),
            pltpu.VMEM((_N2P + 32,), f32),
            pltpu.VMEM((_N2P + 32,), f32),
            pltpu.VMEM((_N2P + 32,), i32),
            pltpu.VMEM((1024,), i32),
            pltpu.VMEM((1024,), i32),
            pltpu.VMEM((_PPW,), f32),
            pltpu.VMEM((_PPW,), f32),
            pltpu.VMEM((_PPW,), f32),
            pltpu.VMEM((_PPW,), f32),
            pltpu.VMEM((_PPW,), f32),
            pltpu.VMEM((_PPW,), f32),
            pltpu.SemaphoreType.DMA,
            pltpu.SemaphoreType.DMA,
            pltpu.VMEM((_QPW,), f32),
            pltpu.VMEM((_QPW,), f32),
            pltpu.VMEM((_QPW,), f32),
            pltpu.VMEM((_CAND,), i32),
            pltpu.VMEM((_QH, _K), i32),
            pltpu.VMEM((_QPW,), i32),
            pltpu.VMEM((_QH, _K), f32),
            pltpu.VMEM((_QH, _K), f32),
            pltpu.VMEM((_QH, _K), f32),
        ],
    )
    def body(p1x_h, p1y_h, p1z_h, p2x_h, p2y_h, p2z_h,
             map_h, nn_h, ox_h, oy_h, oz_h,
             sx_v, sy_v, sz_v, comb_v, cstart_v, hist_v,
             px_s0, py_s0, pz_s0, px_s1, py_s1, pz_s1, sem0, sem1,
             qx_v, qy_v, qz_v,
             cand_v, map_v, nn_v, ox_v, oy_v, oz_v):
        wid = lax.axis_index("c") * 16 + lax.axis_index("s")
        qbase = wid * _QPW
        lanes = lax.iota(i32, 16)
        zeros16 = jnp.zeros((16,), i32)
        onef = jnp.ones((16,), f32)
        zerof = jnp.zeros((16,), f32)
        inf16 = jnp.full((16,), _INF, i32)

        def scalar0(v16):
            return lax.squeeze(lax.slice_in_dim(v16, 0, 1), (0,))

        def scalar_at(v16, r):
            return lax.squeeze(lax.slice_in_dim(v16, r, r + 1), (0,))

        def cell_of(xv, yv, zv):
            cx = jnp.minimum(xv * 10.0, 9.0).astype(i32)
            cy = jnp.minimum(yv * 10.0, 9.0).astype(i32)
            cz = jnp.minimum(zv * 10.0, 9.0).astype(i32)
            return cx * 100 + cy * 10 + cz, cx, cy, cz

        def zero_hist(h, carry):
            hist_v[pl.ds(h * 16, 16)] = zeros16
            return carry

        lax.fori_loop(0, 64, zero_hist, 0)

        bufs = ((px_s0, py_s0, pz_s0), (px_s1, py_s1, pz_s1))
        sems = (sem0, sem1)
        n_slices = _N2P // _PPW

        def issue(s, b):
            return [pltpu.async_copy(src.at[pl.ds(s * _PPW, _PPW)], dst,
                                     sems[b])
                    for src, dst in zip((p2x_h, p2y_h, p2z_h), bufs[b])]

        def run_pass(make_chunk):
            hs = issue(0, 0)
            for s in range(n_slices):
                b = s % 2
                for h in hs:
                    h.wait()
                if s + 1 < n_slices:
                    hs = issue(s + 1, 1 - b)
                bx, by, bz = bufs[b]
                lax.fori_loop(0, _PPW // 16, make_chunk(bx, by, bz, s), 0)

        def p1_chunk(bx, by, bz, s):
            def chunk(c, carry2):
                xv = bx[pl.ds(c * 16, 16)]
                yv = by[pl.ds(c * 16, 16)]
                zv = bz[pl.ds(c * 16, 16)]
                cid, _, _, _ = cell_of(xv, yv, zv)
                rk, is_last = plsc.scan_count(cid)
                old = plsc.load_gather(hist_v, [cid])
                plsc.store_scatter(hist_v, [cid], old + rk, mask=is_last)
                return carry2
            return chunk

        run_pass(p1_chunk)

        def prefix(h, carry):
            ch = hist_v[pl.ds(h * 16, 16)]
            inc = plsc.cumsum(ch)
            cstart_v[pl.ds(h * 16, 16)] = (carry + inc) - ch
            return carry + scalar_at(inc, 15)

        lax.fori_loop(0, 64, prefix, jnp.int32(0))
        lax.fori_loop(0, 64, zero_hist, 0)

        def p2_chunk(bx, by, bz, s):
            def chunk(c, carry2):
                xv = bx[pl.ds(c * 16, 16)]
                yv = by[pl.ds(c * 16, 16)]
                zv = bz[pl.ds(c * 16, 16)]
                cid, _, _, _ = cell_of(xv, yv, zv)
                rk, is_last = plsc.scan_count(cid)
                old = plsc.load_gather(hist_v, [cid])
                base = plsc.load_gather(cstart_v, [cid])
                pos = (base + old) + (rk - 1)
                gidx = (s * _PPW + c * 16) + lanes
                plsc.store_scatter(hist_v, [cid], old + rk, mask=is_last)
                plsc.store_scatter(sx_v, [pos], xv)
                plsc.store_scatter(sy_v, [pos], yv)
                plsc.store_scatter(sz_v, [pos], zv)
                plsc.store_scatter(comb_v, [pos], (gidx << 15) | pos)
                return carry2
            return chunk

        run_pass(p2_chunk)

        farf = jnp.full((16,), 1e6, f32)
        for t in range(2):
            sx_v[pl.ds(_N2P + t * 16, 16)] = farf
            sy_v[pl.ds(_N2P + t * 16, 16)] = farf
            sz_v[pl.ds(_N2P + t * 16, 16)] = farf
            comb_v[pl.ds(_N2P + t * 16, 16)] = inf16

        pltpu.sync_copy(p1x_h.at[pl.ds(qbase, _QPW)], qx_v)
        pltpu.sync_copy(p1y_h.at[pl.ds(qbase, _QPW)], qy_v)
        pltpu.sync_copy(p1z_h.at[pl.ds(qbase, _QPW)], qz_v)

        dxv = lanes // 3 - 1
        dyv = lanes % 3 - 1
        lane_lt9 = lanes < 9

        for half in range(2):
            def per_query(i, carry):
                qi = half * _QH + i
                isplat = jnp.full((16,), qi, i32)
                qx = plsc.load_gather(qx_v, [isplat])
                qy = plsc.load_gather(qy_v, [isplat])
                qz = plsc.load_gather(qz_v, [isplat])
                _, cx, cy, cz = cell_of(qx, qy, qz)
                rowx = cx + dxv
                rowy = cy + dyv
                colxl = rowx.astype(f32) * 0.1
                dxm = jnp.maximum(jnp.maximum(colxl - qx, qx - (colxl + 0.1)),
                                  0.0)
                colyl = rowy.astype(f32) * 0.1
                dym = jnp.maximum(jnp.maximum(colyl - qy, qy - (colyl + 0.1)),
                                  0.0)
                rem = (_R2 + 1e-5 - dxm * dxm) - dym * dym
                czf = cz.astype(f32) * 0.1
                dzlo = qz - czf
                dzhi = (czf + 0.1) - qz
                cz0 = jnp.maximum(cz - jnp.where(dzlo * dzlo <= rem, 1, 0), 0)
                cz1 = jnp.minimum(cz + jnp.where(dzhi * dzhi <= rem, 1, 0), 9)
                okrow = ((rowx >= 0) & (rowx <= 9) & (rowy >= 0)
                         & (rowy <= 9) & lane_lt9 & (rem >= 0.0))
                okrow = okrow & jnp.full((16,), qbase + qi < _N1, jnp.bool_)
                cidr = rowx * 100 + rowy * 10
                cid0 = jnp.where(okrow, cidr + cz0, 0)
                cid1p = jnp.where(okrow, (cidr + cz1) + 1, 0)
                sv = plsc.load_gather(cstart_v, [cid0])
                ev = plsc.load_gather(cstart_v, [cid1p])
                lenv = jnp.where(okrow, ev - sv, 0)

                def seg_chunk(st_r, ln_r):
                    def chunkq(c, cnt):
                        off = st_r + c * 16
                        sxv = sx_v[pl.ds(off, 16)]
                        syv = sy_v[pl.ds(off, 16)]
                        szv = sz_v[pl.ds(off, 16)]
                        comb = comb_v[pl.ds(off, 16)]
                        dx = qx - sxv
                        dy = qy - syv
                        dz = qz - szv
                        d2 = dx * dx + dy * dy
                        d2 = d2 + dz * dz
                        within = d2 <= _R2
                        cntc = jnp.minimum(cnt, _CAND - 16)
                        plsc.store_compressed(cand_v.at[pl.ds(cntc, 16)],
                                              comb, mask=within)
                        c16 = plsc.all_reduce_population_count(within)
                        return cnt + scalar0(c16)
                    return chunkq

                cnt = jnp.int32(0)
                for r in range(9):
                    st_r = scalar_at(sv, r)
                    ln_r = scalar_at(lenv, r)
                    nch = (ln_r + 15) // 16
                    cnt = lax.fori_loop(0, nch, seg_chunk(st_r, ln_r), cnt)

                def select(c, b):
                    b0, b1 = b
                    ch = cand_v[pl.ds(c * 16, 16)]
                    ch = jnp.where((lanes + c * 16) < cnt, ch, inf16)
                    ch = lax.sort(ch)
                    rb = lax.rev(ch, (0,))
                    b0n = lax.sort(jnp.minimum(b0, rb))
                    x = lax.sort(jnp.maximum(b0, rb))
                    rx = lax.rev(x, (0,))
                    b1n = lax.sort(jnp.minimum(b1, rx))
                    return b0n, b1n

                ch0 = jnp.where(lanes < cnt, cand_v[pl.ds(0, 16)], inf16)
                b0 = lax.sort(ch0)
                ch1 = jnp.where((lanes + 16) < cnt, cand_v[pl.ds(16, 16)],
                                inf16)
                ch1 = lax.sort(ch1)
                rb = lax.rev(ch1, (0,))
                b1 = lax.sort(jnp.maximum(b0, rb))
                b0 = lax.sort(jnp.minimum(b0, rb))
                nsel = (jnp.minimum(cnt, _CAND) + 15) // 16
                b0, b1 = lax.fori_loop(2, nsel, select, (b0, b1))

                nn_s = jnp.minimum(cnt, _K)
                nn_splat = jnp.full((16,), nn_s, i32)
                plsc.store_scatter(nn_v, [isplat], nn_splat, mask=lanes == 0)
                for cc, b in enumerate((b0, b1)):
                    validm = (lanes + cc * 16) < nn_splat
                    sidxo = jnp.where(validm, b >> 15, 0)
                    poso = jnp.where(validm, b & 32767, 0)
                    vf = jnp.where(validm, onef, zerof)
                    map_v[i, pl.ds(cc * 16, 16)] = sidxo
                    ox_v[i, pl.ds(cc * 16, 16)] = \
                        plsc.load_gather(sx_v, [poso]) * vf
                    oy_v[i, pl.ds(cc * 16, 16)] = \
                        plsc.load_gather(sy_v, [poso]) * vf
                    oz_v[i, pl.ds(cc * 16, 16)] = \
                        plsc.load_gather(sz_v, [poso]) * vf
                return carry

            lax.fori_loop(0, _QH, per_query, 0)
            hb = qbase + half * _QH
            pltpu.sync_copy(map_v, map_h.at[pl.ds(hb, _QH)])
            pltpu.sync_copy(ox_v, ox_h.at[pl.ds(hb, _QH)])
            pltpu.sync_copy(oy_v, oy_h.at[pl.ds(hb, _QH)])
            pltpu.sync_copy(oz_v, oz_h.at[pl.ds(hb, _QH)])
        pltpu.sync_copy(nn_v, nn_h.at[pl.ds(qbase, _QPW)])

    return body(p1x, p1y, p1z, p2x, p2y, p2z)


def kernel(points1, points2):
    p1 = points1[0]
    p2 = points2[0]
    p1p = jnp.pad(p1, ((0, _NQPAD - _N1), (0, 0)))
    p2p = jnp.pad(p2, ((0, _N2P - _N2), (0, 0)), constant_values=1e6)
    p1x, p1y, p1z = p1p[:, 0], p1p[:, 1], p1p[:, 2]
    p2x, p2y, p2z = p2p[:, 0], p2p[:, 1], p2p[:, 2]
    mp, nn, ox, oy, oz = _ball_query_sc(p1x, p1y, p1z, p2x, p2y, p2z)
    mapping = mp[:_N1].reshape(1, _N1, _K)
    num_neighbors = nn[:_N1].reshape(1, _N1)
    outputs = jnp.stack([ox[:_N1], oy[:_N1], oz[:_N1]], axis=-1)
    outputs = outputs.reshape(1, _N1, _K, 3)
    return mapping, num_neighbors, outputs


# speedup vs baseline: 1.1583x; 1.0270x over previous
import functools

import jax
import jax.numpy as jnp
from jax import lax
from jax.experimental import pallas as pl
from jax.experimental.pallas import tpu as pltpu
from jax.experimental.pallas import tpu_sc as plsc

_K = 32
_N1 = 10000
_N2 = 20000
_NW = 32
_QPW = 320
_NQPAD = _NW * _QPW
_QH = _QPW // 2
_R2 = 0.1 * 0.1
_N2P = 20480
_PPW = _N2P // 16
_CAND = 2048
_INF = 0x7FFFFFFF


def _ball_query_sc(p1x, p1y, p1z, p2x, p2y, p2z):
    f32 = jnp.float32
    i32 = jnp.int32
    mesh = plsc.VectorSubcoreMesh(core_axis_name="c", subcore_axis_name="s")

    @functools.partial(
        pl.kernel,
        out_type=[
            jax.ShapeDtypeStruct((_NQPAD, _K), i32),
            jax.ShapeDtypeStruct((_NQPAD,), i32),
            jax.ShapeDtypeStruct((_NQPAD, _K), f32),
            jax.ShapeDtypeStruct((_NQPAD, _K), f32),
            jax.ShapeDtypeStruct((_NQPAD, _K), f32),
        ],
        mesh=mesh,
        compiler_params=pltpu.CompilerParams(needs_layout_passes=False,
                                             use_tc_tiling_on_sc=False),
        scratch_types=[
            pltpu.VMEM((_N2P + 32,), f32),
            pltpu.VMEM((_N2P + 32,), f32),
            pltpu.VMEM((_N2P + 32,), f32),
            pltpu.VMEM((_N2P + 32,), i32),
            pltpu.VMEM((1024,), i32),
            pltpu.VMEM((1024,), i32),
            pltpu.VMEM((_PPW,), f32),
            pltpu.VMEM((_PPW,), f32),
            pltpu.VMEM((_PPW,), f32),
            pltpu.VMEM((_PPW,), f32),
            pltpu.VMEM((_PPW,), f32),
            pltpu.VMEM((_PPW,), f32),
            pltpu.SemaphoreType.DMA,
            pltpu.SemaphoreType.DMA,
            pltpu.VMEM((_QPW,), f32),
            pltpu.VMEM((_QPW,), f32),
            pltpu.VMEM((_QPW,), f32),
            pltpu.VMEM((_CAND,), i32),
            pltpu.VMEM((_QH, _K), i32),
            pltpu.VMEM((_QPW,), i32),
            pltpu.VMEM((_QH, _K), f32),
            pltpu.VMEM((_QH, _K), f32),
            pltpu.VMEM((_QH, _K), f32),
        ],
    )
    def body(p1x_h, p1y_h, p1z_h, p2x_h, p2y_h, p2z_h,
             map_h, nn_h, ox_h, oy_h, oz_h,
             sx_v, sy_v, sz_v, comb_v, cstart_v, hist_v,
             px_s0, py_s0, pz_s0, px_s1, py_s1, pz_s1, sem0, sem1,
             qx_v, qy_v, qz_v,
             cand_v, map_v, nn_v, ox_v, oy_v, oz_v):
        wid = lax.axis_index("c") * 16 + lax.axis_index("s")
        qbase = wid * _QPW
        lanes = lax.iota(i32, 16)
        zeros16 = jnp.zeros((16,), i32)
        onef = jnp.ones((16,), f32)
        zerof = jnp.zeros((16,), f32)
        inf16 = jnp.full((16,), _INF, i32)

        def scalar0(v16):
            return lax.squeeze(lax.slice_in_dim(v16, 0, 1), (0,))

        def scalar_at(v16, r):
            return lax.squeeze(lax.slice_in_dim(v16, r, r + 1), (0,))

        def cell_of(xv, yv, zv):
            cx = jnp.minimum(xv * 10.0, 9.0).astype(i32)
            cy = jnp.minimum(yv * 10.0, 9.0).astype(i32)
            cz = jnp.minimum(zv * 10.0, 9.0).astype(i32)
            return cx * 100 + cy * 10 + cz, cx, cy, cz

        def zero_hist(h, carry):
            hist_v[pl.ds(h * 16, 16)] = zeros16
            return carry

        lax.fori_loop(0, 64, zero_hist, 0)

        bufs = ((px_s0, py_s0, pz_s0), (px_s1, py_s1, pz_s1))
        sems = (sem0, sem1)
        n_slices = _N2P // _PPW

        def issue(s, b):
            return [pltpu.async_copy(src.at[pl.ds(s * _PPW, _PPW)], dst,
                                     sems[b])
                    for src, dst in zip((p2x_h, p2y_h, p2z_h), bufs[b])]

        def run_pass(make_chunk):
            hs = issue(0, 0)
            for s in range(n_slices):
                b = s % 2
                for h in hs:
                    h.wait()
                if s + 1 < n_slices:
                    hs = issue(s + 1, 1 - b)
                bx, by, bz = bufs[b]
                lax.fori_loop(0, _PPW // 16, make_chunk(bx, by, bz, s), 0)

        ones16 = jnp.ones((16,), i32)

        def p1_chunk(bx, by, bz, s):
            def chunk(c, carry2):
                xv = bx[pl.ds(c * 16, 16)]
                yv = by[pl.ds(c * 16, 16)]
                zv = bz[pl.ds(c * 16, 16)]
                cid, _, _, _ = cell_of(xv, yv, zv)
                plsc.addupdate_scatter(hist_v, [cid], ones16)
                return carry2
            return chunk

        run_pass(p1_chunk)

        def prefix(h, carry):
            ch = hist_v[pl.ds(h * 16, 16)]
            inc = plsc.cumsum(ch)
            cstart_v[pl.ds(h * 16, 16)] = (carry + inc) - ch
            return carry + scalar_at(inc, 15)

        lax.fori_loop(0, 64, prefix, jnp.int32(0))
        lax.fori_loop(0, 64, zero_hist, 0)

        def p2_chunk(bx, by, bz, s):
            def chunk(c, carry2):
                xv = bx[pl.ds(c * 16, 16)]
                yv = by[pl.ds(c * 16, 16)]
                zv = bz[pl.ds(c * 16, 16)]
                cid, _, _, _ = cell_of(xv, yv, zv)
                rk, is_last = plsc.scan_count(cid)
                old = plsc.load_gather(hist_v, [cid])
                base = plsc.load_gather(cstart_v, [cid])
                pos = (base + old) + (rk - 1)
                gidx = (s * _PPW + c * 16) + lanes
                plsc.store_scatter(hist_v, [cid], old + rk, mask=is_last)
                plsc.store_scatter(sx_v, [pos], xv)
                plsc.store_scatter(sy_v, [pos], yv)
                plsc.store_scatter(sz_v, [pos], zv)
                plsc.store_scatter(comb_v, [pos], (gidx << 15) | pos)
                return carry2
            return chunk

        run_pass(p2_chunk)

        farf = jnp.full((16,), 1e6, f32)
        for t in range(2):
            sx_v[pl.ds(_N2P + t * 16, 16)] = farf
            sy_v[pl.ds(_N2P + t * 16, 16)] = farf
            sz_v[pl.ds(_N2P + t * 16, 16)] = farf
            comb_v[pl.ds(_N2P + t * 16, 16)] = inf16

        pltpu.sync_copy(p1x_h.at[pl.ds(qbase, _QPW)], qx_v)
        pltpu.sync_copy(p1y_h.at[pl.ds(qbase, _QPW)], qy_v)
        pltpu.sync_copy(p1z_h.at[pl.ds(qbase, _QPW)], qz_v)

        dxv = lanes // 3 - 1
        dyv = lanes % 3 - 1
        lane_lt9 = lanes < 9

        for half in range(2):
            def per_query(i, carry):
                qi = half * _QH + i
                isplat = jnp.full((16,), qi, i32)
                qx = plsc.load_gather(qx_v, [isplat])
                qy = plsc.load_gather(qy_v, [isplat])
                qz = plsc.load_gather(qz_v, [isplat])
                _, cx, cy, cz = cell_of(qx, qy, qz)
                rowx = cx + dxv
                rowy = cy + dyv
                colxl = rowx.astype(f32) * 0.1
                dxm = jnp.maximum(jnp.maximum(colxl - qx, qx - (colxl + 0.1)),
                                  0.0)
                colyl = rowy.astype(f32) * 0.1
                dym = jnp.maximum(jnp.maximum(colyl - qy, qy - (colyl + 0.1)),
                                  0.0)
                rem = (_R2 + 1e-5 - dxm * dxm) - dym * dym
                czf = cz.astype(f32) * 0.1
                dzlo = qz - czf
                dzhi = (czf + 0.1) - qz
                cz0 = jnp.maximum(cz - jnp.where(dzlo * dzlo <= rem, 1, 0), 0)
                cz1 = jnp.minimum(cz + jnp.where(dzhi * dzhi <= rem, 1, 0), 9)
                okrow = ((rowx >= 0) & (rowx <= 9) & (rowy >= 0)
                         & (rowy <= 9) & lane_lt9 & (rem >= 0.0))
                okrow = okrow & jnp.full((16,), qbase + qi < _N1, jnp.bool_)
                cidr = rowx * 100 + rowy * 10
                cid0 = jnp.where(okrow, cidr + cz0, 0)
                cid1p = jnp.where(okrow, (cidr + cz1) + 1, 0)
                sv = plsc.load_gather(cstart_v, [cid0])
                ev = plsc.load_gather(cstart_v, [cid1p])
                lenv = jnp.where(okrow, ev - sv, 0)

                def seg_chunk(st_r, ln_r):
                    def chunkq(c, cnt):
                        off = st_r + c * 16
                        sxv = sx_v[pl.ds(off, 16)]
                        syv = sy_v[pl.ds(off, 16)]
                        szv = sz_v[pl.ds(off, 16)]
                        comb = comb_v[pl.ds(off, 16)]
                        dx = qx - sxv
                        dy = qy - syv
                        dz = qz - szv
                        d2 = dx * dx + dy * dy
                        d2 = d2 + dz * dz
                        within = d2 <= _R2
                        cntc = jnp.minimum(cnt, _CAND - 16)
                        plsc.store_compressed(cand_v.at[pl.ds(cntc, 16)],
                                              comb, mask=within)
                        c16 = plsc.all_reduce_population_count(within)
                        return cnt + scalar0(c16)
                    return chunkq

                cnt = jnp.int32(0)
                for r in range(9):
                    st_r = scalar_at(sv, r)
                    ln_r = scalar_at(lenv, r)
                    nch = (ln_r + 15) // 16
                    cnt = lax.fori_loop(0, nch, seg_chunk(st_r, ln_r), cnt)

                def select(c, b):
                    b0, b1 = b
                    ch = cand_v[pl.ds(c * 16, 16)]
                    ch = jnp.where((lanes + c * 16) < cnt, ch, inf16)
                    ch = lax.sort(ch)
                    rb = lax.rev(ch, (0,))
                    b0n = lax.sort(jnp.minimum(b0, rb))
                    x = lax.sort(jnp.maximum(b0, rb))
                    rx = lax.rev(x, (0,))
                    b1n = lax.sort(jnp.minimum(b1, rx))
                    return b0n, b1n

                ch0 = jnp.where(lanes < cnt, cand_v[pl.ds(0, 16)], inf16)
                b0 = lax.sort(ch0)
                ch1 = jnp.where((lanes + 16) < cnt, cand_v[pl.ds(16, 16)],
                                inf16)
                ch1 = lax.sort(ch1)
                rb = lax.rev(ch1, (0,))
                b1 = lax.sort(jnp.maximum(b0, rb))
                b0 = lax.sort(jnp.minimum(b0, rb))
                nsel = (jnp.minimum(cnt, _CAND) + 15) // 16
                b0, b1 = lax.fori_loop(2, nsel, select, (b0, b1))

                nn_s = jnp.minimum(cnt, _K)
                nn_splat = jnp.full((16,), nn_s, i32)
                plsc.store_scatter(nn_v, [isplat], nn_splat, mask=lanes == 0)
                for cc, b in enumerate((b0, b1)):
                    validm = (lanes + cc * 16) < nn_splat
                    sidxo = jnp.where(validm, b >> 15, 0)
                    poso = jnp.where(validm, b & 32767, 0)
                    vf = jnp.where(validm, onef, zerof)
                    map_v[i, pl.ds(cc * 16, 16)] = sidxo
                    ox_v[i, pl.ds(cc * 16, 16)] = \
                        plsc.load_gather(sx_v, [poso]) * vf
                    oy_v[i, pl.ds(cc * 16, 16)] = \
                        plsc.load_gather(sy_v, [poso]) * vf
                    oz_v[i, pl.ds(cc * 16, 16)] = \
                        plsc.load_gather(sz_v, [poso]) * vf
                return carry

            lax.fori_loop(0, _QH, per_query, 0)
            hb = qbase + half * _QH
            pltpu.sync_copy(map_v, map_h.at[pl.ds(hb, _QH)])
            pltpu.sync_copy(ox_v, ox_h.at[pl.ds(hb, _QH)])
            pltpu.sync_copy(oy_v, oy_h.at[pl.ds(hb, _QH)])
            pltpu.sync_copy(oz_v, oz_h.at[pl.ds(hb, _QH)])
        pltpu.sync_copy(nn_v, nn_h.at[pl.ds(qbase, _QPW)])

    return body(p1x, p1y, p1z, p2x, p2y, p2z)


def kernel(points1, points2):
    p1 = points1[0]
    p2 = points2[0]
    p1p = jnp.pad(p1, ((0, _NQPAD - _N1), (0, 0)))
    p2p = jnp.pad(p2, ((0, _N2P - _N2), (0, 0)), constant_values=1e6)
    p1x, p1y, p1z = p1p[:, 0], p1p[:, 1], p1p[:, 2]
    p2x, p2y, p2z = p2p[:, 0], p2p[:, 1], p2p[:, 2]
    mp, nn, ox, oy, oz = _ball_query_sc(p1x, p1y, p1z, p2x, p2y, p2z)
    mapping = mp[:_N1].reshape(1, _N1, _K)
    num_neighbors = nn[:_N1].reshape(1, _N1)
    outputs = jnp.stack([ox[:_N1], oy[:_N1], oz[:_N1]], axis=-1)
    outputs = outputs.reshape(1, _N1, _K, 3)
    return mapping, num_neighbors, outputs
